# baseline (speedup 1.0000x reference)
import functools

import jax
import jax.numpy as jnp
from jax.experimental import pallas as pl
from jax.experimental.pallas import tpu as pltpu

_NUM_CLASSES = 4
_PROTO_MOMENTUM = 0.9
_CONF_THRESH = 0.8
_LAMBDA_INTRA = 1.0
_LAMBDA_INTER = 0.1
_MARGIN_M = 1.0

_CHUNK = 8192
_NSPLIT = 4


def _body(*refs, nb, nc, kcls):
    (pred_ref, label_ref, islab_ref, proto_ref, pinit_ref) = refs[:5]
    feat_refs = refs[5:5 + _NSPLIT]
    out_ref = refs[5 + _NSPLIT]
    sums_s, cnt_s, s2_s = refs[6 + _NSPLIT:]
    b = pl.program_id(0)
    i = pl.program_id(1)

    @pl.when((b == 0) & (i == 0))
    def _init():
        sums_s[...] = jnp.zeros_like(sums_s)
        cnt_s[...] = jnp.zeros_like(cnt_s)
        s2_s[...] = jnp.zeros_like(s2_s)

    p = pred_ref[0]
    lbl = label_ref[0]
    islab = islab_ref[0, 0, 0]

    conf = p[0:1]
    cls = jnp.zeros_like(lbl)
    for k in range(1, kcls):
        pk = p[k:k + 1]
        better = pk > conf
        conf = jnp.where(better, pk, conf)
        cls = jnp.where(better, k, cls)

    mask = (conf > _CONF_THRESH) & (cls > 0)
    mask &= (cls == lbl) | (islab < 0.5)
    w = mask.astype(jnp.float32)

    kv = jax.lax.broadcasted_iota(jnp.int32, (_NUM_CLASSES, 1), 0) + 1
    w4 = jnp.where(cls == kv, w, 0.0)

    dot = functools.partial(jax.lax.dot_general,
                            precision=jax.lax.Precision.DEFAULT,
                            preferred_element_type=jnp.float32)
    parts = []
    s2 = None
    for fr in feat_refs:
        f = fr[0]
        parts.append(dot(w4, f, (((1,), (1,)), ((), ()))))
        fsq = jnp.sum(f * f, axis=0, keepdims=True)
        s2 = fsq if s2 is None else s2 + fsq
    sums_s[...] += jnp.concatenate(parts, axis=1)
    cnt_s[...] += jnp.sum(w4, axis=1, keepdims=True)
    s2_s[...] += dot(w4, s2, (((1,), (1,)), ((), ())))

    @pl.when((b == nb - 1) & (i == nc - 1))
    def _finalize():
        sums = sums_s[...]
        counts = cnt_s[...]
        s2sum = s2_s[...]
        proto = proto_ref[...]
        pinit = pinit_ref[...] > 0.5

        present = counts > 0.0
        new_proto = sums / jnp.maximum(counts, 1.0)
        updated = jnp.where(present,
                            jnp.where(pinit,
                                      _PROTO_MOMENTUM * proto
                                      + (1.0 - _PROTO_MOMENTUM) * new_proto,
                                      new_proto),
                            proto)
        ini = pinit | present

        un2 = jnp.sum(updated * updated, axis=1, keepdims=True)
        total = (jnp.sum(s2sum)
                 - 2.0 * jnp.sum(sums * updated)
                 + jnp.sum(counts * un2))
        vp = jnp.sum(counts)
        loss_intra = jnp.where(vp > 0.0, total / jnp.maximum(vp, 1.0), 0.0)

        gram = jax.lax.dot_general(updated, updated,
                                   (((1,), (1,)), ((), ())),
                                   precision=jax.lax.Precision.HIGHEST,
                                   preferred_element_type=jnp.float32)
        d2m = un2 + jnp.reshape(un2, (1, _NUM_CLASSES)) - 2.0 * gram
        dist = jnp.sqrt(jnp.maximum(d2m, 0.0) + 1e-12)
        r = jax.lax.broadcasted_iota(jnp.int32, (_NUM_CLASSES, _NUM_CLASSES), 0)
        c = jax.lax.broadcasted_iota(jnp.int32, (_NUM_CLASSES, _NUM_CLASSES), 1)
        pair_valid = (ini & jnp.reshape(ini, (1, _NUM_CLASSES)) & (c > r))
        pen = jnp.maximum(_MARGIN_M - dist, 0.0) ** 2
        pvf = pair_valid.astype(jnp.float32)
        n_pairs = jnp.sum(pvf)
        loss_inter = jnp.where(n_pairs > 0.0,
                               jnp.sum(pen * pvf) / jnp.maximum(n_pairs, 1.0),
                               0.0)
        loss = _LAMBDA_INTRA * loss_intra + _LAMBDA_INTER * loss_inter
        out_ref[...] = jnp.reshape(loss, (1, 1))


def kernel(feat, pred, label, is_labelled, prototypes, prototype_initialized):
    B, C, H, W, D = feat.shape
    K = pred.shape[1]
    M = H * W * D
    chunk = min(_CHUNK, M)
    nc = M // chunk

    feat3 = feat.reshape(B, C, M)
    pred3 = pred.reshape(B, K, M)
    label3 = label.reshape(B, 1, M)
    islab = is_labelled.astype(jnp.float32).reshape(B, 1, 1)
    pinit = prototype_initialized.astype(jnp.float32).reshape(_NUM_CLASSES, 1)

    cs = C // _NSPLIT
    feat_specs = [
        pl.BlockSpec((1, cs, chunk), functools.partial(
            lambda b, i, jj: (b, jj, i), jj=j))
        for j in range(_NSPLIT)
    ]
    out = pl.pallas_call(
        functools.partial(_body, nb=B, nc=nc, kcls=K),
        grid=(B, nc),
        in_specs=[
            pl.BlockSpec((1, K, chunk), lambda b, i: (b, 0, i)),
            pl.BlockSpec((1, 1, chunk), lambda b, i: (b, 0, i)),
            pl.BlockSpec((1, 1, 1), lambda b, i: (b, 0, 0)),
            pl.BlockSpec((_NUM_CLASSES, C), lambda b, i: (0, 0)),
            pl.BlockSpec((_NUM_CLASSES, 1), lambda b, i: (0, 0)),
        ] + feat_specs,
        out_specs=pl.BlockSpec((1, 1), lambda b, i: (0, 0)),
        out_shape=jax.ShapeDtypeStruct((1, 1), jnp.float32),
        scratch_shapes=[
            pltpu.VMEM((_NUM_CLASSES, C), jnp.float32),
            pltpu.VMEM((_NUM_CLASSES, 1), jnp.float32),
            pltpu.VMEM((_NUM_CLASSES, 1), jnp.float32),
        ],
    )(pred3, label3, islab, prototypes, pinit,
      *([feat3] * _NSPLIT))
    return out.reshape(())


# speedup vs baseline: 1.0725x; 1.0725x over previous
import functools

import jax
import jax.numpy as jnp
from jax.experimental import pallas as pl
from jax.experimental.pallas import tpu as pltpu

_NUM_CLASSES = 4
_PROTO_MOMENTUM = 0.9
_CONF_THRESH = 0.8
_LAMBDA_INTRA = 1.0
_LAMBDA_INTER = 0.1
_MARGIN_M = 1.0

_CHUNK = 32768
_NSPLIT = 1


def _body(*refs, nb, nc, kcls):
    (pred_ref, label_ref, islab_ref, proto_ref, pinit_ref) = refs[:5]
    feat_refs = refs[5:5 + _NSPLIT]
    out_ref = refs[5 + _NSPLIT]
    sums_s, cnt_s, s2_s = refs[6 + _NSPLIT:]
    b = pl.program_id(0)
    i = pl.program_id(1)

    @pl.when((b == 0) & (i == 0))
    def _init():
        sums_s[...] = jnp.zeros_like(sums_s)
        cnt_s[...] = jnp.zeros_like(cnt_s)
        s2_s[...] = jnp.zeros_like(s2_s)

    p = pred_ref[0]
    lbl = label_ref[0]
    islab = islab_ref[0, 0, 0]

    conf = p[0:1]
    cls = jnp.zeros_like(lbl)
    for k in range(1, kcls):
        pk = p[k:k + 1]
        better = pk > conf
        conf = jnp.where(better, pk, conf)
        cls = jnp.where(better, k, cls)

    mask = (conf > _CONF_THRESH) & (cls > 0)
    mask &= (cls == lbl) | (islab < 0.5)
    w = mask.astype(jnp.float32)

    kv = jax.lax.broadcasted_iota(jnp.int32, (_NUM_CLASSES, 1), 0) + 1
    w4 = jnp.where(cls == kv, w, 0.0)

    dot = functools.partial(jax.lax.dot_general,
                            precision=jax.lax.Precision.DEFAULT,
                            preferred_element_type=jnp.float32)
    parts = []
    s2 = None
    for fr in feat_refs:
        f = fr[0]
        parts.append(dot(w4, f, (((1,), (1,)), ((), ()))))
        fsq = jnp.sum(f * f, axis=0, keepdims=True)
        s2 = fsq if s2 is None else s2 + fsq
    sums_s[...] += parts[0] if _NSPLIT == 1 else jnp.concatenate(parts, axis=1)
    cnt_s[...] += jnp.sum(w4, axis=1, keepdims=True)
    s2_s[...] += dot(w4, s2, (((1,), (1,)), ((), ())))

    @pl.when((b == nb - 1) & (i == nc - 1))
    def _finalize():
        sums = sums_s[...]
        counts = cnt_s[...]
        s2sum = s2_s[...]
        proto = proto_ref[...]
        pinit = pinit_ref[...] > 0.5

        present = counts > 0.0
        new_proto = sums / jnp.maximum(counts, 1.0)
        updated = jnp.where(present,
                            jnp.where(pinit,
                                      _PROTO_MOMENTUM * proto
                                      + (1.0 - _PROTO_MOMENTUM) * new_proto,
                                      new_proto),
                            proto)
        ini = pinit | present

        un2 = jnp.sum(updated * updated, axis=1, keepdims=True)
        total = (jnp.sum(s2sum)
                 - 2.0 * jnp.sum(sums * updated)
                 + jnp.sum(counts * un2))
        vp = jnp.sum(counts)
        loss_intra = jnp.where(vp > 0.0, total / jnp.maximum(vp, 1.0), 0.0)

        gram = jax.lax.dot_general(updated, updated,
                                   (((1,), (1,)), ((), ())),
                                   precision=jax.lax.Precision.HIGHEST,
                                   preferred_element_type=jnp.float32)
        d2m = un2 + jnp.reshape(un2, (1, _NUM_CLASSES)) - 2.0 * gram
        dist = jnp.sqrt(jnp.maximum(d2m, 0.0) + 1e-12)
        r = jax.lax.broadcasted_iota(jnp.int32, (_NUM_CLASSES, _NUM_CLASSES), 0)
        c = jax.lax.broadcasted_iota(jnp.int32, (_NUM_CLASSES, _NUM_CLASSES), 1)
        pair_valid = (ini & jnp.reshape(ini, (1, _NUM_CLASSES)) & (c > r))
        pen = jnp.maximum(_MARGIN_M - dist, 0.0) ** 2
        pvf = pair_valid.astype(jnp.float32)
        n_pairs = jnp.sum(pvf)
        loss_inter = jnp.where(n_pairs > 0.0,
                               jnp.sum(pen * pvf) / jnp.maximum(n_pairs, 1.0),
                               0.0)
        loss = _LAMBDA_INTRA * loss_intra + _LAMBDA_INTER * loss_inter
        out_ref[...] = jnp.reshape(loss, (1, 1))


def kernel(feat, pred, label, is_labelled, prototypes, prototype_initialized):
    B, C, H, W, D = feat.shape
    K = pred.shape[1]
    M = H * W * D
    chunk = min(_CHUNK, M)
    nc = M // chunk

    feat3 = feat.reshape(B, C, M)
    pred3 = pred.reshape(B, K, M)
    label3 = label.reshape(B, 1, M)
    islab = is_labelled.astype(jnp.float32).reshape(B, 1, 1)
    pinit = prototype_initialized.astype(jnp.float32).reshape(_NUM_CLASSES, 1)

    cs = C // _NSPLIT
    feat_specs = [
        pl.BlockSpec((1, cs, chunk), functools.partial(
            lambda b, i, jj: (b, jj, i), jj=j))
        for j in range(_NSPLIT)
    ]
    out = pl.pallas_call(
        functools.partial(_body, nb=B, nc=nc, kcls=K),
        grid=(B, nc),
        in_specs=[
            pl.BlockSpec((1, K, chunk), lambda b, i: (b, 0, i)),
            pl.BlockSpec((1, 1, chunk), lambda b, i: (b, 0, i)),
            pl.BlockSpec((1, 1, 1), lambda b, i: (b, 0, 0)),
            pl.BlockSpec((_NUM_CLASSES, C), lambda b, i: (0, 0)),
            pl.BlockSpec((_NUM_CLASSES, 1), lambda b, i: (0, 0)),
        ] + feat_specs,
        out_specs=pl.BlockSpec((1, 1), lambda b, i: (0, 0)),
        out_shape=jax.ShapeDtypeStruct((1, 1), jnp.float32),
        scratch_shapes=[
            pltpu.VMEM((_NUM_CLASSES, C), jnp.float32),
            pltpu.VMEM((_NUM_CLASSES, 1), jnp.float32),
            pltpu.VMEM((_NUM_CLASSES, 1), jnp.float32),
        ],
    )(pred3, label3, islab, prototypes, pinit,
      *([feat3] * _NSPLIT))
    return out.reshape(())
